# baseline (speedup 1.0000x reference)
import jax
import jax.numpy as jnp
from jax.experimental import pallas as pl
from jax.experimental.pallas import tpu as pltpu


def _pool_body(x_ref, o_ref):
    rows_per, e, b = x_ref.shape
    w_out = o_ref.shape[1]
    cols_per = e // w_out
    s = x_ref[...].sum(axis=0)
    t = s.reshape(w_out, cols_per, b).sum(axis=1)
    o_ref[...] = (t * (1.0 / float(rows_per * cols_per))).reshape(1, w_out, b)


@jax.jit
def _adaptive_pool(x):
    B, N, E = x.shape
    H, W = 4, 8
    rows_per = N // H

    xt = jnp.transpose(x, (1, 2, 0))

    cost = pl.CostEstimate(
        flops=B * N * E,
        transcendentals=0,
        bytes_accessed=B * N * E * 4 + B * H * W * 4,
    )
    out_t = pl.pallas_call(
        _pool_body,
        out_shape=jax.ShapeDtypeStruct((H, W, B), jnp.float32),
        grid=(H,),
        in_specs=[pl.BlockSpec((rows_per, E, B), lambda i: (i, 0, 0))],
        out_specs=pl.BlockSpec((1, W, B), lambda i: (i, 0, 0)),
        compiler_params=pltpu.CompilerParams(
            dimension_semantics=("arbitrary",),
        ),
        cost_estimate=cost,
    )(xt)
    return jnp.transpose(out_t.reshape(H * W, B)).astype(x.dtype)


def kernel(x):
    return _adaptive_pool(x)
